# baseline (speedup 1.0000x reference)
import functools

import jax
import jax.numpy as jnp
from jax import lax
from jax.experimental import pallas as pl
from jax.experimental.pallas import tpu as pltpu
from jax.experimental.pallas import tpu_sc as plsc

_N_E = 8192
_E_DIM = 64
_N_TOK = 9216
_BETA = 0.25

_TM = 512
_TN = 1024
_GT = _N_TOK // _TM
_GN = _N_E // _TN


def _vq_body(z_ref, e_ref, idx_ref, loss_ref, minv_ref, mini_ref):
    t = pl.program_id(0)
    n = pl.program_id(1)

    @pl.when(n == 0)
    def _init():
        minv_ref[...] = jnp.full((_TM, 1), jnp.inf, jnp.float32)
        mini_ref[...] = jnp.zeros((_TM, 1), jnp.int32)

    @pl.when((t == 0) & (n == 0))
    def _init_loss():
        loss_ref[...] = jnp.zeros_like(loss_ref)

    z = z_ref[...]
    e = e_ref[...]
    zsq = jnp.sum(z * z, axis=1, keepdims=True)
    esq = jnp.sum(e * e, axis=1)[None, :]
    mm = lax.dot_general(z.astype(jnp.bfloat16), e,
                         (((1,), (1,)), ((), ())),
                         preferred_element_type=jnp.float32)
    d = (zsq + esq) - 2.0 * mm

    lmin = jnp.min(d, axis=1, keepdims=True)
    cols = lax.broadcasted_iota(jnp.int32, (_TM, _TN), 1)
    lidx = jnp.min(jnp.where(d == lmin, cols, _TN), axis=1,
                   keepdims=True) + n * _TN

    pred = lmin < minv_ref[...]
    mini_ref[...] = jnp.where(pred, lidx, mini_ref[...])
    minv_ref[...] = jnp.where(pred, lmin, minv_ref[...])

    @pl.when(n == _GN - 1)
    def _finish():
        idx_ref[...] = mini_ref[...].reshape((_TM,))
        loss_ref[...] += jnp.sum(minv_ref[...]).reshape(1, 1)

    @pl.when((n == _GN - 1) & (t == _GT - 1))
    def _scale():
        loss_ref[...] = loss_ref[...] * ((1.0 + _BETA) / (_N_TOK * _E_DIM))


def _vq_argmin(z_flat, emb):
    return pl.pallas_call(
        _vq_body,
        grid=(_GT, _GN),
        in_specs=[
            pl.BlockSpec((_TM, _E_DIM), lambda t, n: (t, 0)),
            pl.BlockSpec((_TN, _E_DIM), lambda t, n: (n, 0)),
        ],
        out_specs=[
            pl.BlockSpec((_TM,), lambda t, n: (t,)),
            pl.BlockSpec((1, 1), lambda t, n: (0, 0)),
        ],
        out_shape=[
            jax.ShapeDtypeStruct((_N_TOK,), jnp.int32),
            jax.ShapeDtypeStruct((1, 1), jnp.float32),
        ],
        scratch_shapes=[
            pltpu.VMEM((_TM, 1), jnp.float32),
            pltpu.VMEM((_TM, 1), jnp.int32),
        ],
    )(z_flat, emb)


@functools.cache
def _make_sc_gather():
    info = plsc.get_sparse_core_info()
    nw = info.num_cores * info.num_subcores
    b_per_w = _N_TOK // nw
    ch = 96
    n_ch = b_per_w // ch
    mesh = plsc.VectorSubcoreMesh(core_axis_name="c", subcore_axis_name="s")

    @functools.partial(
        pl.kernel,
        mesh=mesh,
        out_type=jax.ShapeDtypeStruct((_N_TOK, _E_DIM), jnp.float32),
        scratch_types=[
            pltpu.VMEM((ch,), jnp.int32),
            pltpu.VMEM((ch, _E_DIM), jnp.float32),
            pltpu.SemaphoreType.DMA,
        ],
        compiler_params=pltpu.CompilerParams(use_tc_tiling_on_sc=False),
    )
    def gather_k(table_hbm, idx_hbm, out_hbm, idx_v, rows_v, sem):
        wid = lax.axis_index("s") * info.num_cores + lax.axis_index("c")
        base = wid * b_per_w
        for j in range(n_ch):
            off = base + j * ch
            pltpu.sync_copy(idx_hbm.at[pl.ds(off, ch)], idx_v)
            pltpu.async_copy(table_hbm.at[idx_v], rows_v, sem).wait()
            pltpu.sync_copy(rows_v, out_hbm.at[pl.ds(off, ch)])

    return gather_k


def kernel(z, embedding_weight):
    z_flat = z.reshape(-1, _E_DIM)
    idx, loss = _vq_argmin(z_flat, embedding_weight)
    z_q = _make_sc_gather()(embedding_weight, idx)
    return (z_q.reshape(z.shape), loss[0, 0], idx)


# speedup vs baseline: 1.0488x; 1.0488x over previous
import functools

import jax
import jax.numpy as jnp
from jax import lax
from jax.experimental import pallas as pl
from jax.experimental.pallas import tpu as pltpu
from jax.experimental.pallas import tpu_sc as plsc

_N_E = 8192
_E_DIM = 64
_N_TOK = 9216
_BETA = 0.25

_TM = 512
_TN = 1024
_GT = _N_TOK // _TM
_GN = _N_E // _TN


def _vq_body(z_ref, e_ref, idx_ref, loss_ref, mink_ref, minn_ref):
    t = pl.program_id(0)
    n = pl.program_id(1)

    @pl.when(n == 0)
    def _init():
        mink_ref[...] = jnp.full((_TM, 1), jnp.int32(0x7F7FFFFF))
        minn_ref[...] = jnp.zeros((_TM, 1), jnp.int32)

    @pl.when((t == 0) & (n == 0))
    def _init_loss():
        loss_ref[...] = jnp.zeros_like(loss_ref)

    z = z_ref[...]
    e = e_ref[...]
    zsq = jnp.sum(z * z, axis=1, keepdims=True)
    esq = jnp.sum(e * e, axis=1)[None, :]
    mm2 = lax.dot_general((-2.0 * z).astype(jnp.bfloat16), e,
                          (((1,), (1,)), ((), ())),
                          preferred_element_type=jnp.float32)
    d = (zsq + esq) + mm2

    cols = lax.broadcasted_iota(jnp.int32, (_TM, _TN), 1)
    key = (lax.bitcast_convert_type(d, jnp.int32) & ~jnp.int32(1023)) | cols
    lkey = jnp.min(key, axis=1, keepdims=True)

    pred = lkey < mink_ref[...]
    minn_ref[...] = jnp.where(pred, n, minn_ref[...])
    mink_ref[...] = jnp.where(pred, lkey, mink_ref[...])

    @pl.when(n == _GN - 1)
    def _finish():
        k = mink_ref[...]
        idx = minn_ref[...] * _TN + (k & jnp.int32(1023))
        idx_ref[...] = idx.reshape((_TM,))
        dmin = lax.bitcast_convert_type(k & ~jnp.int32(1023), jnp.float32)
        loss_ref[...] += jnp.sum(dmin).reshape(1, 1)

    @pl.when((n == _GN - 1) & (t == _GT - 1))
    def _scale():
        loss_ref[...] = loss_ref[...] * ((1.0 + _BETA) / (_N_TOK * _E_DIM))


def _vq_argmin(z_flat, emb):
    return pl.pallas_call(
        _vq_body,
        grid=(_GT, _GN),
        in_specs=[
            pl.BlockSpec((_TM, _E_DIM), lambda t, n: (t, 0)),
            pl.BlockSpec((_TN, _E_DIM), lambda t, n: (n, 0)),
        ],
        out_specs=[
            pl.BlockSpec((_TM,), lambda t, n: (t,)),
            pl.BlockSpec((1, 1), lambda t, n: (0, 0)),
        ],
        out_shape=[
            jax.ShapeDtypeStruct((_N_TOK,), jnp.int32),
            jax.ShapeDtypeStruct((1, 1), jnp.float32),
        ],
        scratch_shapes=[
            pltpu.VMEM((_TM, 1), jnp.int32),
            pltpu.VMEM((_TM, 1), jnp.int32),
        ],
    )(z_flat, emb)


@functools.cache
def _make_sc_gather():
    info = plsc.get_sparse_core_info()
    nw = info.num_cores * info.num_subcores
    b_per_w = _N_TOK // nw
    ch = 96
    n_ch = b_per_w // ch
    mesh = plsc.VectorSubcoreMesh(core_axis_name="c", subcore_axis_name="s")

    @functools.partial(
        pl.kernel,
        mesh=mesh,
        out_type=jax.ShapeDtypeStruct((_N_TOK, _E_DIM), jnp.float32),
        scratch_types=[
            pltpu.VMEM((ch,), jnp.int32),
            pltpu.VMEM((ch, _E_DIM), jnp.float32),
            pltpu.SemaphoreType.DMA,
        ],
        compiler_params=pltpu.CompilerParams(use_tc_tiling_on_sc=False),
    )
    def gather_k(table_hbm, idx_hbm, out_hbm, idx_v, rows_v, sem):
        wid = lax.axis_index("s") * info.num_cores + lax.axis_index("c")
        base = wid * b_per_w
        for j in range(n_ch):
            off = base + j * ch
            pltpu.sync_copy(idx_hbm.at[pl.ds(off, ch)], idx_v)
            pltpu.async_copy(table_hbm.at[idx_v], rows_v, sem).wait()
            pltpu.sync_copy(rows_v, out_hbm.at[pl.ds(off, ch)])

    return gather_k


def kernel(z, embedding_weight):
    z_flat = z.reshape(-1, _E_DIM)
    idx, loss = _vq_argmin(z_flat, embedding_weight)
    z_q = _make_sc_gather()(embedding_weight, idx)
    return (z_q.reshape(z.shape), loss[0, 0], idx)


# speedup vs baseline: 1.0580x; 1.0087x over previous
import functools

import jax
import jax.numpy as jnp
from jax import lax
from jax.experimental import pallas as pl
from jax.experimental.pallas import tpu as pltpu
from jax.experimental.pallas import tpu_sc as plsc

_N_E = 8192
_E_DIM = 64
_N_TOK = 9216
_BETA = 0.25

_TM = 512
_TN = 1024
_GT = _N_TOK // _TM
_GN = _N_E // _TN


def _vq_body(z_ref, e_ref, idx_ref, loss_ref, mink_ref, minn_ref):
    t = pl.program_id(0)
    n = pl.program_id(1)

    @pl.when(n == 0)
    def _init():
        mink_ref[...] = jnp.full((_TM, 1), jnp.int32(0x7F7FFFFF))
        minn_ref[...] = jnp.zeros((_TM, 1), jnp.int32)

    @pl.when((t == 0) & (n == 0))
    def _init_loss():
        loss_ref[...] = jnp.zeros_like(loss_ref)

    z = z_ref[...]
    e = e_ref[...]
    zsq = jnp.sum(z * z, axis=1, keepdims=True)
    esq = jnp.sum(e * e, axis=1)[None, :]
    mm2 = lax.dot_general((-2.0 * z).astype(jnp.bfloat16), e,
                          (((1,), (1,)), ((), ())),
                          preferred_element_type=jnp.float32)
    d = (zsq + esq) + mm2

    cols = lax.broadcasted_iota(jnp.int32, (_TM, _TN), 1)
    key = (lax.bitcast_convert_type(d, jnp.int32) & ~jnp.int32(1023)) | cols
    lkey = jnp.min(key, axis=1, keepdims=True)

    pred = lkey < mink_ref[...]
    minn_ref[...] = jnp.where(pred, n, minn_ref[...])
    mink_ref[...] = jnp.where(pred, lkey, mink_ref[...])

    @pl.when(n == _GN - 1)
    def _finish():
        k = mink_ref[...]
        idx = minn_ref[...] * _TN + (k & jnp.int32(1023))
        idx_ref[...] = idx.reshape((_TM,))
        dmin = lax.bitcast_convert_type(k & ~jnp.int32(1023), jnp.float32)
        loss_ref[...] += jnp.sum(dmin).reshape(1, 1)

    @pl.when((n == _GN - 1) & (t == _GT - 1))
    def _scale():
        loss_ref[...] = loss_ref[...] * ((1.0 + _BETA) / (_N_TOK * _E_DIM))


def _vq_argmin(z_flat, emb):
    return pl.pallas_call(
        _vq_body,
        grid=(_GT, _GN),
        in_specs=[
            pl.BlockSpec((_TM, _E_DIM), lambda t, n: (t, 0)),
            pl.BlockSpec((_TN, _E_DIM), lambda t, n: (n, 0)),
        ],
        out_specs=[
            pl.BlockSpec((_TM,), lambda t, n: (t,)),
            pl.BlockSpec((1, 1), lambda t, n: (0, 0)),
        ],
        out_shape=[
            jax.ShapeDtypeStruct((_N_TOK,), jnp.int32),
            jax.ShapeDtypeStruct((1, 1), jnp.float32),
        ],
        scratch_shapes=[
            pltpu.VMEM((_TM, 1), jnp.int32),
            pltpu.VMEM((_TM, 1), jnp.int32),
        ],
    )(z_flat, emb)


@functools.cache
def _make_sc_gather():
    info = plsc.get_sparse_core_info()
    nw = info.num_cores * info.num_subcores
    b_per_w = _N_TOK // nw
    ch = 96
    n_ch = b_per_w // ch
    mesh = plsc.VectorSubcoreMesh(core_axis_name="c", subcore_axis_name="s")

    @functools.partial(
        pl.kernel,
        mesh=mesh,
        out_type=jax.ShapeDtypeStruct((_N_TOK, _E_DIM), jnp.float32),
        scratch_types=[
            pltpu.VMEM((b_per_w,), jnp.int32),
            [pltpu.VMEM((ch, _E_DIM), jnp.float32) for _ in range(3)],
            pltpu.SemaphoreType.DMA,
            pltpu.SemaphoreType.DMA,
        ],
        compiler_params=pltpu.CompilerParams(use_tc_tiling_on_sc=False),
    )
    def gather_k(table_hbm, idx_hbm, out_hbm, idx_v, rows, gsem, wsem):
        wid = lax.axis_index("s") * info.num_cores + lax.axis_index("c")
        base = wid * b_per_w
        pltpu.sync_copy(idx_hbm.at[pl.ds(base, b_per_w)], idx_v)
        gathers = [
            pltpu.async_copy(table_hbm.at[idx_v.at[pl.ds(j * ch, ch)]],
                             rows[j], gsem)
            for j in range(n_ch)
        ]
        writes = []
        for j in range(n_ch):
            gathers[j].wait()
            writes.append(pltpu.async_copy(
                rows[j], out_hbm.at[pl.ds(base + j * ch, ch)], wsem))
        for w in writes:
            w.wait()

    return gather_k


def kernel(z, embedding_weight):
    z_flat = z.reshape(-1, _E_DIM)
    idx, loss = _vq_argmin(z_flat, embedding_weight)
    z_q = _make_sc_gather()(embedding_weight, idx)
    return (z_q.reshape(z.shape), loss[0, 0], idx)


# speedup vs baseline: 1.3259x; 1.2533x over previous
import functools

import jax
import jax.numpy as jnp
from jax import lax
from jax.experimental import pallas as pl
from jax.experimental.pallas import tpu as pltpu
from jax.experimental.pallas import tpu_sc as plsc

_N_E = 8192
_E_DIM = 64
_N_TOK = 9216
_BETA = 0.25

_TM = 512
_TN = 2048
_GT = _N_TOK // _TM
_GN = _N_E // _TN


def _vq_body(z_ref, e_ref, idx_ref, loss_ref, mink_ref, minn_ref):
    t = pl.program_id(0)
    n = pl.program_id(1)

    @pl.when(n == 0)
    def _init():
        mink_ref[...] = jnp.full((_TM, 1), jnp.int32(0x7F7FFFFF))
        minn_ref[...] = jnp.zeros((_TM, 1), jnp.int32)

    @pl.when((t == 0) & (n == 0))
    def _init_loss():
        loss_ref[...] = jnp.zeros_like(loss_ref)

    z = z_ref[...]
    e = e_ref[...]
    zsq = jnp.sum(z * z, axis=1, keepdims=True)
    esq = jnp.sum(e * e, axis=1)[None, :]
    mm2 = lax.dot_general((-2.0 * z).astype(jnp.bfloat16), e,
                          (((1,), (1,)), ((), ())),
                          preferred_element_type=jnp.float32)
    d = (zsq + esq) + mm2

    cols = lax.broadcasted_iota(jnp.int32, (_TM, _TN), 1)
    key = (lax.bitcast_convert_type(d, jnp.int32) & ~jnp.int32(2047)) | cols
    lkey = jnp.min(key, axis=1, keepdims=True)

    pred = lkey < mink_ref[...]
    minn_ref[...] = jnp.where(pred, n, minn_ref[...])
    mink_ref[...] = jnp.where(pred, lkey, mink_ref[...])

    @pl.when(n == _GN - 1)
    def _finish():
        k = mink_ref[...]
        idx = minn_ref[...] * _TN + (k & jnp.int32(2047))
        idx_ref[...] = idx.reshape((_TM,))
        dmin = lax.bitcast_convert_type(k & ~jnp.int32(2047), jnp.float32)
        loss_ref[...] += jnp.sum(dmin).reshape(1, 1)

    @pl.when((n == _GN - 1) & (t == _GT - 1))
    def _scale():
        loss_ref[...] = loss_ref[...] * ((1.0 + _BETA) / (_N_TOK * _E_DIM))


def _vq_argmin(z_flat, emb):
    return pl.pallas_call(
        _vq_body,
        grid=(_GT, _GN),
        in_specs=[
            pl.BlockSpec((_TM, _E_DIM), lambda t, n: (t, 0)),
            pl.BlockSpec((_TN, _E_DIM), lambda t, n: (n, 0)),
        ],
        out_specs=[
            pl.BlockSpec((_TM,), lambda t, n: (t,)),
            pl.BlockSpec((1, 1), lambda t, n: (0, 0)),
        ],
        out_shape=[
            jax.ShapeDtypeStruct((_N_TOK,), jnp.int32),
            jax.ShapeDtypeStruct((1, 1), jnp.float32),
        ],
        scratch_shapes=[
            pltpu.VMEM((_TM, 1), jnp.int32),
            pltpu.VMEM((_TM, 1), jnp.int32),
        ],
    )(z_flat, emb)


@functools.cache
def _make_sc_gather():
    info = plsc.get_sparse_core_info()
    nw = info.num_cores * info.num_subcores
    b_per_w = _N_TOK // nw
    ch = 96
    n_ch = b_per_w // ch
    mesh = plsc.VectorSubcoreMesh(core_axis_name="c", subcore_axis_name="s")

    @functools.partial(
        pl.kernel,
        mesh=mesh,
        out_type=jax.ShapeDtypeStruct((_N_TOK, _E_DIM), jnp.float32),
        scratch_types=[
            pltpu.VMEM((b_per_w,), jnp.int32),
            [pltpu.VMEM((ch, _E_DIM), jnp.float32) for _ in range(3)],
            pltpu.SemaphoreType.DMA,
            pltpu.SemaphoreType.DMA,
        ],
        compiler_params=pltpu.CompilerParams(use_tc_tiling_on_sc=False),
    )
    def gather_k(table_hbm, idx_hbm, out_hbm, idx_v, rows, gsem, wsem):
        wid = lax.axis_index("s") * info.num_cores + lax.axis_index("c")
        base = wid * b_per_w
        pltpu.sync_copy(idx_hbm.at[pl.ds(base, b_per_w)], idx_v)
        gathers = [
            pltpu.async_copy(table_hbm.at[idx_v.at[pl.ds(j * ch, ch)]],
                             rows[j], gsem)
            for j in range(n_ch)
        ]
        writes = []
        for j in range(n_ch):
            gathers[j].wait()
            writes.append(pltpu.async_copy(
                rows[j], out_hbm.at[pl.ds(base + j * ch, ch)], wsem))
        for w in writes:
            w.wait()

    return gather_k


def kernel(z, embedding_weight):
    z_flat = z.reshape(-1, _E_DIM)
    idx, loss = _vq_argmin(z_flat, embedding_weight)
    z_q = _make_sc_gather()(embedding_weight, idx)
    return (z_q.reshape(z.shape), loss[0, 0], idx)
